# Initial kernel scaffold; baseline (speedup 1.0000x reference)
import functools

import jax
import jax.numpy as jnp
from jax import lax
from jax.experimental import pallas as pl
from jax.experimental.pallas import tpu as pltpu
from jax.experimental.pallas import tpu_sc as plsc

N = 10000
E = 320000
D = 128

NC = 2
NS = 16
NW = NC * NS
EPW = E // NW
RPT = N // NS
K = 80
NCHUNK = EPW // K

_mesh = plsc.VectorSubcoreMesh(core_axis_name="c", subcore_axis_name="s")


@functools.partial(
    pl.kernel,
    out_type=jax.ShapeDtypeStruct((NC * N, 16), jnp.float32),
    mesh=_mesh,
    scratch_types=[
        pltpu.VMEM((K,), jnp.int32),
        pltpu.VMEM((K, 16), jnp.float32),
        pltpu.VMEM((125, 16), jnp.float32),
        pltpu.VMEM_SHARED((N, 16), jnp.float32),
    ],
)
def _deg_kernel(dst_hbm, degp_hbm, dstbuf, ones, zbuf, deg2):
    c = lax.axis_index("c")
    s = lax.axis_index("s")
    wid = c * NS + s
    base = wid * EPW

    def fill(i, _):
        ones[i, :] = jnp.ones((16,), jnp.float32)
        return 0

    lax.fori_loop(0, K, fill, 0)

    def fillz(i, _):
        zbuf[i, :] = jnp.zeros((16,), jnp.float32)
        return 0

    lax.fori_loop(0, 125, fillz, 0)

    for j in range(5):
        pltpu.sync_copy(zbuf, deg2.at[pl.ds(s * RPT + j * 125, 125)])
    plsc.subcore_barrier()

    def body(i, _):
        pltpu.sync_copy(dst_hbm.at[pl.ds(base + i * K, K)], dstbuf)
        pltpu.sync_copy(ones, deg2.at[dstbuf], add=True)
        return 0

    lax.fori_loop(0, NCHUNK, body, 0)
    plsc.subcore_barrier()

    pltpu.sync_copy(
        deg2.at[pl.ds(s * RPT, RPT)],
        degp_hbm.at[pl.ds(c * N + s * RPT, RPT)],
    )


def _xw_body(x_ref, w_ref, degp_ref, xs_ref):
    deg = degp_ref[0:N, 0:1] + degp_ref[N : 2 * N, 0:1] + 1.0
    dinv = lax.rsqrt(deg)
    xw = jnp.dot(x_ref[...], w_ref[...], preferred_element_type=jnp.float32)
    xs_ref[...] = dinv * xw


_xw_kernel = pl.pallas_call(
    _xw_body,
    out_shape=jax.ShapeDtypeStruct((N, D), jnp.float32),
)


@functools.partial(
    pl.kernel,
    out_type=jax.ShapeDtypeStruct((NC * N, D), jnp.float32),
    mesh=_mesh,
    scratch_types=[
        pltpu.VMEM((K,), jnp.int32),
        pltpu.VMEM((K,), jnp.int32),
        pltpu.VMEM((K, D), jnp.float32),
        pltpu.VMEM_SHARED((N, D), jnp.float32),
        pltpu.SemaphoreType.DMA,
    ],
)
def _edge_kernel(xs_hbm, src_hbm, dst_hbm, accp_hbm, srcbuf, dstbuf, rows,
                 acc, sem):
    c = lax.axis_index("c")
    s = lax.axis_index("s")
    wid = c * NS + s
    base = wid * EPW

    def zrows(i, _):
        rows[i // 8, pl.ds((i % 8) * 16, 16)] = jnp.zeros((16,), jnp.float32)
        return 0

    lax.fori_loop(0, K * 8, zrows, 0)

    for j in range(7):
        pltpu.sync_copy(rows, acc.at[pl.ds(s * RPT + j * K, K)])
    pltpu.sync_copy(rows.at[pl.ds(0, 65)], acc.at[pl.ds(s * RPT + 7 * K, 65)])
    plsc.subcore_barrier()

    def body(i, _):
        off = base + i * K
        pltpu.sync_copy(src_hbm.at[pl.ds(off, K)], srcbuf)
        pltpu.sync_copy(dst_hbm.at[pl.ds(off, K)], dstbuf)
        pltpu.async_copy(xs_hbm.at[srcbuf], rows, sem).wait()
        pltpu.sync_copy(rows, acc.at[dstbuf], add=True)
        return 0

    lax.fori_loop(0, NCHUNK, body, 0)
    plsc.subcore_barrier()

    pltpu.sync_copy(
        acc.at[pl.ds(s * RPT, RPT)],
        accp_hbm.at[pl.ds(c * N + s * RPT, RPT)],
    )


def _comb_body(accp_ref, xs_ref, degp_ref, b_ref, o_ref):
    deg = degp_ref[0:N, 0:1] + degp_ref[N : 2 * N, 0:1] + 1.0
    dinv = lax.rsqrt(deg)
    o_ref[...] = (
        dinv * (accp_ref[0:N, :] + accp_ref[N : 2 * N, :] + xs_ref[...])
        + b_ref[...]
    )


_comb_kernel = pl.pallas_call(
    _comb_body,
    out_shape=jax.ShapeDtypeStruct((N, D), jnp.float32),
)


def kernel(x, edge_index, W, b):
    src = edge_index[0]
    dst = edge_index[1]
    degp = _deg_kernel(dst)
    xs = _xw_kernel(x, W, degp)
    accp = _edge_kernel(xs, src, dst)
    return _comb_kernel(accp, xs, degp, b.reshape(1, D))


# speedup vs baseline: 17.9958x; 17.9958x over previous
import functools

import jax
import jax.numpy as jnp
from jax import lax
from jax.experimental import pallas as pl
from jax.experimental.pallas import tpu as pltpu
from jax.experimental.pallas import tpu_sc as plsc

N = 10000
E = 320000
D = 128

NC = 2
NS = 16
NW = NC * NS
EPW = E // NW
NP = 10240
RPT = NP // NS
K = 80
NCHUNK = EPW // K

_mesh = plsc.VectorSubcoreMesh(core_axis_name="c", subcore_axis_name="s")


@functools.partial(
    pl.kernel,
    out_type=jax.ShapeDtypeStruct((NC * NP, 16), jnp.float32),
    mesh=_mesh,
    scratch_types=[
        pltpu.VMEM((K,), jnp.int32),
        pltpu.VMEM((K, 16), jnp.float32),
        pltpu.VMEM((128, 16), jnp.float32),
        pltpu.VMEM_SHARED((NP, 16), jnp.float32),
    ],
)
def _deg_kernel(dst_hbm, degp_hbm, dstbuf, ones, zbuf, deg2):
    c = lax.axis_index("c")
    s = lax.axis_index("s")
    wid = c * NS + s
    base = wid * EPW

    def fill(i, _):
        ones[i, :] = jnp.ones((16,), jnp.float32)
        return 0

    lax.fori_loop(0, K, fill, 0)

    def fillz(i, _):
        zbuf[i, :] = jnp.zeros((16,), jnp.float32)
        return 0

    lax.fori_loop(0, 128, fillz, 0)

    for j in range(5):
        pltpu.sync_copy(zbuf, deg2.at[pl.ds(s * RPT + j * 128, 128)])
    plsc.subcore_barrier()

    def body(i, _):
        pltpu.sync_copy(dst_hbm.at[pl.ds(base + i * K, K)], dstbuf)
        pltpu.sync_copy(ones, deg2.at[dstbuf], add=True)
        return 0

    lax.fori_loop(0, NCHUNK, body, 0)
    plsc.subcore_barrier()

    pltpu.sync_copy(
        deg2.at[pl.ds(s * RPT, RPT)],
        degp_hbm.at[pl.ds(c * NP + s * RPT, RPT)],
    )


def _xw_body(x_ref, w_ref, degp_ref, xs_ref):
    deg = degp_ref[0:N, 0:1] + degp_ref[NP : NP + N, 0:1] + 1.0
    dinv = lax.rsqrt(deg)
    xw = jnp.dot(x_ref[...], w_ref[...], preferred_element_type=jnp.float32)
    xs_ref[...] = dinv * xw


_xw_kernel = pl.pallas_call(
    _xw_body,
    out_shape=jax.ShapeDtypeStruct((N, D), jnp.float32),
)


@functools.partial(
    pl.kernel,
    out_type=jax.ShapeDtypeStruct((NC * NP, D), jnp.float32),
    mesh=_mesh,
    scratch_types=[
        pltpu.VMEM((K,), jnp.int32),
        pltpu.VMEM((K,), jnp.int32),
        pltpu.VMEM((K, D), jnp.float32),
        pltpu.VMEM_SHARED((NP, D), jnp.float32),
        pltpu.SemaphoreType.DMA,
    ],
)
def _edge_kernel(xs_hbm, src_hbm, dst_hbm, accp_hbm, srcbuf, dstbuf, rows,
                 acc, sem):
    c = lax.axis_index("c")
    s = lax.axis_index("s")
    wid = c * NS + s
    base = wid * EPW

    def zrows(i, _):
        rows[i // 8, pl.ds((i % 8) * 16, 16)] = jnp.zeros((16,), jnp.float32)
        return 0

    lax.fori_loop(0, K * 8, zrows, 0)

    for j in range(8):
        pltpu.sync_copy(rows, acc.at[pl.ds(s * RPT + j * K, K)])
    plsc.subcore_barrier()

    def body(i, _):
        off = base + i * K
        pltpu.sync_copy(src_hbm.at[pl.ds(off, K)], srcbuf)
        pltpu.sync_copy(dst_hbm.at[pl.ds(off, K)], dstbuf)
        pltpu.async_copy(xs_hbm.at[srcbuf], rows, sem).wait()
        pltpu.sync_copy(rows, acc.at[dstbuf], add=True)
        return 0

    lax.fori_loop(0, NCHUNK, body, 0)
    plsc.subcore_barrier()

    pltpu.sync_copy(
        acc.at[pl.ds(s * RPT, RPT)],
        accp_hbm.at[pl.ds(c * NP + s * RPT, RPT)],
    )


def _comb_body(accp_ref, xs_ref, degp_ref, b_ref, o_ref):
    deg = degp_ref[0:N, 0:1] + degp_ref[NP : NP + N, 0:1] + 1.0
    dinv = lax.rsqrt(deg)
    o_ref[...] = (
        dinv * (accp_ref[0:N, :] + accp_ref[NP : NP + N, :] + xs_ref[...])
        + b_ref[...]
    )


_comb_kernel = pl.pallas_call(
    _comb_body,
    out_shape=jax.ShapeDtypeStruct((N, D), jnp.float32),
)


def kernel(x, edge_index, W, b):
    src = edge_index[0]
    dst = edge_index[1]
    degp = _deg_kernel(dst)
    xs = _xw_kernel(x, W, degp)
    accp = _edge_kernel(xs, src, dst)
    return _comb_kernel(accp, xs, degp, b.reshape(1, D))


# speedup vs baseline: 39.2831x; 2.1829x over previous
import functools

import jax
import jax.numpy as jnp
from jax import lax
from jax.experimental import pallas as pl
from jax.experimental.pallas import tpu as pltpu
from jax.experimental.pallas import tpu_sc as plsc

N = 10000
E = 320000
D = 128

NC = 2
NS = 16
NW = NC * NS
EPW = E // NW
NP = 10240
RPT = NP // NS

KD = 125
NCD = EPW // KD
NBD = 4
NRD = NCD // NBD

K = 80
NCHUNK = EPW // K
NB = 4
NRINGS = (NCHUNK - 1) // NB

_mesh = plsc.VectorSubcoreMesh(core_axis_name="c", subcore_axis_name="s")


@functools.partial(
    pl.kernel,
    out_type=jax.ShapeDtypeStruct((NC * NP, 16), jnp.float32),
    mesh=_mesh,
    scratch_types=[
        pltpu.VMEM((NCD, KD), jnp.int32),
        pltpu.VMEM((KD, 16), jnp.float32),
        pltpu.VMEM((128, 16), jnp.float32),
        pltpu.VMEM_SHARED((NP, 16), jnp.float32),
    ]
    + [pltpu.SemaphoreType.DMA] * NBD,
)
def _deg_kernel(dst_hbm, degp_hbm, dstb, ones, zbuf, deg2, *ssem):
    c = lax.axis_index("c")
    s = lax.axis_index("s")
    wid = c * NS + s

    def fill(i, _):
        ones[i, :] = jnp.ones((16,), jnp.float32)
        return 0

    lax.fori_loop(0, KD, fill, 0)

    def fillz(i, _):
        zbuf[i, :] = jnp.zeros((16,), jnp.float32)
        return 0

    lax.fori_loop(0, 128, fillz, 0)

    for j in range(5):
        pltpu.sync_copy(zbuf, deg2.at[pl.ds(s * RPT + j * 128, 128)])
    pltpu.sync_copy(dst_hbm.at[wid], dstb)
    plsc.subcore_barrier()

    def scat(b, i):
        pltpu.async_copy(ones, deg2.at[dstb.at[i]], ssem[b])

    def swait(b):
        pltpu.make_async_copy(ones, deg2.at[dstb.at[0]], ssem[b]).wait()

    for b in range(NBD):
        scat(b, b)

    def ring(g, _):
        for b in range(NBD):
            swait(b)
            scat(b, (g + 1) * NBD + b)
        return 0

    lax.fori_loop(0, NRD - 1, ring, 0)
    for b in range(NBD):
        swait(b)
    plsc.subcore_barrier()

    pltpu.sync_copy(
        deg2.at[pl.ds(s * RPT, RPT)],
        degp_hbm.at[pl.ds(c * NP + s * RPT, RPT)],
    )


def _xw_body(x_ref, w_ref, degp_ref, xs_ref):
    deg = degp_ref[0:N, 0:1] + degp_ref[NP : NP + N, 0:1] + 1.0
    dinv = lax.rsqrt(deg)
    xw = jnp.dot(x_ref[...], w_ref[...], preferred_element_type=jnp.float32)
    xs_ref[...] = dinv * xw


_xw_kernel = pl.pallas_call(
    _xw_body,
    out_shape=jax.ShapeDtypeStruct((N, D), jnp.float32),
)


@functools.partial(
    pl.kernel,
    out_type=jax.ShapeDtypeStruct((NC * NP, D), jnp.float32),
    mesh=_mesh,
    scratch_types=[pltpu.VMEM((K,), jnp.int32)] * NB
    + [pltpu.VMEM((K,), jnp.int32)] * NB
    + [pltpu.VMEM((K, D), jnp.float32)] * NB
    + [pltpu.VMEM_SHARED((NP, D), jnp.float32)]
    + [pltpu.SemaphoreType.DMA] * (3 * NB),
)
def _edge_kernel(xs_hbm, src_hbm, dst_hbm, accp_hbm, *rest):
    srcb = rest[:NB]
    dstb = rest[NB : 2 * NB]
    rows = rest[2 * NB : 3 * NB]
    acc = rest[3 * NB]
    isem = rest[3 * NB + 1 : 3 * NB + 1 + NB]
    gsem = rest[3 * NB + 1 + NB : 3 * NB + 1 + 2 * NB]
    ssem = rest[3 * NB + 1 + 2 * NB :]
    c = lax.axis_index("c")
    s = lax.axis_index("s")
    wid = c * NS + s
    base = wid * EPW

    def zrows(i, _):
        rows[0][i // 8, pl.ds((i % 8) * 16, 16)] = jnp.zeros(
            (16,), jnp.float32
        )
        return 0

    lax.fori_loop(0, K * 8, zrows, 0)

    for j in range(8):
        pltpu.sync_copy(rows[0], acc.at[pl.ds(s * RPT + j * K, K)])
    plsc.subcore_barrier()

    def idx_start(b, i):
        off = base + i * K
        pltpu.async_copy(src_hbm.at[pl.ds(off, K)], srcb[b], isem[b])
        pltpu.async_copy(dst_hbm.at[pl.ds(off, K)], dstb[b], isem[b])

    def iwait(b):
        pltpu.make_async_copy(src_hbm.at[pl.ds(base, K)], srcb[b],
                              isem[b]).wait()
        pltpu.make_async_copy(dst_hbm.at[pl.ds(base, K)], dstb[b],
                              isem[b]).wait()

    def gath(b):
        pltpu.async_copy(xs_hbm.at[srcb[b]], rows[b], gsem[b])

    def gwait(b):
        pltpu.make_async_copy(xs_hbm.at[srcb[b]], rows[b], gsem[b]).wait()

    def scat(b):
        pltpu.async_copy(rows[b], acc.at[dstb[b]], ssem[b])

    def swait(b):
        pltpu.make_async_copy(rows[b], acc.at[dstb[b]], ssem[b]).wait()

    for b in range(NB):
        idx_start(b, b)

    def ring(g, _):
        for b in range(NB):
            iwait(b)
            gath(b)
        for b in range(NB):
            gwait(b)
            scat(b)
        for b in range(NB):
            swait(b)
            idx_start(b, jnp.minimum((g + 1) * NB + b, NCHUNK - 1))
        return 0

    lax.fori_loop(0, NRINGS, ring, 0)
    for b in range(NB):
        iwait(b)
    gath(0)
    gwait(0)
    scat(0)
    swait(0)
    plsc.subcore_barrier()

    pltpu.sync_copy(
        acc.at[pl.ds(s * RPT, RPT)],
        accp_hbm.at[pl.ds(c * NP + s * RPT, RPT)],
    )


def _comb_body(accp_ref, xs_ref, degp_ref, b_ref, o_ref):
    deg = degp_ref[0:N, 0:1] + degp_ref[NP : NP + N, 0:1] + 1.0
    dinv = lax.rsqrt(deg)
    o_ref[...] = (
        dinv * (accp_ref[0:N, :] + accp_ref[NP : NP + N, :] + xs_ref[...])
        + b_ref[...]
    )


_comb_kernel = pl.pallas_call(
    _comb_body,
    out_shape=jax.ShapeDtypeStruct((N, D), jnp.float32),
)


def kernel(x, edge_index, W, b):
    src = edge_index[0]
    dst = edge_index[1]
    dst3 = dst.reshape(NW, NCD, KD)
    degp = _deg_kernel(dst3)
    xs = _xw_kernel(x, W, degp)
    accp = _edge_kernel(xs, src, dst)
    return _comb_kernel(accp, xs, degp, b.reshape(1, D))


# speedup vs baseline: 47.1939x; 1.2014x over previous
import functools

import jax
import jax.numpy as jnp
from jax import lax
from jax.experimental import pallas as pl
from jax.experimental.pallas import tpu as pltpu
from jax.experimental.pallas import tpu_sc as plsc

N = 10000
E = 320000
D = 128

NC = 2
NS = 16
NW = NC * NS
EPW = E // NW
NP = 10240
RPT = NP // NS

KD = 125
NCD = EPW // KD
NBD = 4
NRD = NCD // NBD

K = 80
NCHUNK = EPW // K
NB = 4
NRINGS = (NCHUNK - 1) // NB

_mesh = plsc.VectorSubcoreMesh(core_axis_name="c", subcore_axis_name="s")


@functools.partial(
    pl.kernel,
    out_type=jax.ShapeDtypeStruct((NC * NP, 16), jnp.float32),
    mesh=_mesh,
    scratch_types=[
        pltpu.VMEM((NCD, KD), jnp.int32),
        pltpu.VMEM((KD, 16), jnp.float32),
        pltpu.VMEM((128, 16), jnp.float32),
        pltpu.VMEM_SHARED((NP, 16), jnp.float32),
    ]
    + [pltpu.SemaphoreType.DMA] * NBD,
)
def _deg_kernel(dst_hbm, degp_hbm, dstb, ones, zbuf, deg2, *ssem):
    c = lax.axis_index("c")
    s = lax.axis_index("s")
    wid = c * NS + s

    def fill(i, _):
        ones[i, :] = jnp.ones((16,), jnp.float32)
        return 0

    lax.fori_loop(0, KD, fill, 0)

    def fillz(i, _):
        zbuf[i, :] = jnp.zeros((16,), jnp.float32)
        return 0

    lax.fori_loop(0, 128, fillz, 0)

    for j in range(5):
        pltpu.sync_copy(zbuf, deg2.at[pl.ds(s * RPT + j * 128, 128)])
    pltpu.sync_copy(dst_hbm.at[wid], dstb)
    plsc.subcore_barrier()

    def scat(b, i):
        pltpu.async_copy(ones, deg2.at[dstb.at[i]], ssem[b])

    def swait(b):
        pltpu.make_async_copy(ones, deg2.at[dstb.at[0]], ssem[b]).wait()

    for b in range(NBD):
        scat(b, b)

    def ring(g, _):
        for b in range(NBD):
            swait(b)
            scat(b, (g + 1) * NBD + b)
        return 0

    lax.fori_loop(0, NRD - 1, ring, 0)
    for b in range(NBD):
        swait(b)
    plsc.subcore_barrier()

    pltpu.sync_copy(
        deg2.at[pl.ds(s * RPT, RPT)],
        degp_hbm.at[pl.ds(c * NP + s * RPT, RPT)],
    )


def _xw_body(x_ref, w_ref, degp_ref, xs_ref):
    deg = degp_ref[0:N, 0:1] + degp_ref[NP : NP + N, 0:1] + 1.0
    dinv = lax.rsqrt(deg)
    xw = jnp.dot(x_ref[...], w_ref[...], preferred_element_type=jnp.float32)
    xs_ref[...] = dinv * xw


_xw_kernel = pl.pallas_call(
    _xw_body,
    out_shape=jax.ShapeDtypeStruct((N, D), jnp.float32),
)


@functools.partial(
    pl.kernel,
    out_type=jax.ShapeDtypeStruct((NC * NP, D), jnp.float32),
    mesh=_mesh,
    scratch_types=[pltpu.VMEM((K,), jnp.int32)] * NB
    + [pltpu.VMEM((K,), jnp.int32)] * NB
    + [pltpu.VMEM((K, D), jnp.float32)] * NB
    + [pltpu.VMEM_SHARED((NP, D), jnp.float32)]
    + [pltpu.SemaphoreType.DMA] * (3 * NB),
)
def _edge_kernel(xs_hbm, src_hbm, dst_hbm, accp_hbm, *rest):
    srcb = rest[:NB]
    dstb = rest[NB : 2 * NB]
    rows = rest[2 * NB : 3 * NB]
    acc = rest[3 * NB]
    isem = rest[3 * NB + 1 : 3 * NB + 1 + NB]
    gsem = rest[3 * NB + 1 + NB : 3 * NB + 1 + 2 * NB]
    ssem = rest[3 * NB + 1 + 2 * NB :]
    c = lax.axis_index("c")
    s = lax.axis_index("s")
    wid = c * NS + s
    base = wid * EPW

    def zrows(i, _):
        rows[0][i // 8, pl.ds((i % 8) * 16, 16)] = jnp.zeros(
            (16,), jnp.float32
        )
        return 0

    lax.fori_loop(0, K * 8, zrows, 0)

    for j in range(8):
        pltpu.sync_copy(rows[0], acc.at[pl.ds(s * RPT + j * K, K)])
    plsc.subcore_barrier()

    def idx_start(b, i):
        off = base + i * K
        pltpu.async_copy(src_hbm.at[pl.ds(off, K)], srcb[b], isem[b])
        pltpu.async_copy(dst_hbm.at[pl.ds(off, K)], dstb[b], isem[b])

    def iwait(b):
        pltpu.make_async_copy(src_hbm.at[pl.ds(base, K)], srcb[b],
                              isem[b]).wait()
        pltpu.make_async_copy(dst_hbm.at[pl.ds(base, K)], dstb[b],
                              isem[b]).wait()

    def gath(b):
        pltpu.async_copy(xs_hbm.at[srcb[b]], rows[b], gsem[b])

    def gwait(b):
        pltpu.make_async_copy(xs_hbm.at[srcb[b]], rows[b], gsem[b]).wait()

    def scat(b):
        pltpu.async_copy(rows[b], acc.at[dstb[b]], ssem[b])

    def swait(b):
        pltpu.make_async_copy(rows[b], acc.at[dstb[b]], ssem[b]).wait()

    for b in range(NB):
        idx_start(b, b)
    iwait(0)
    gath(0)
    iwait(1)
    gath(1)

    def ring(g, _):
        for b in range(NB):
            i = g * NB + b
            gwait(b)
            scat(b)
            b2 = (b + 2) % NB
            iwait(b2)
            gath(b2)
            swait(b)
            idx_start(b, i + NB)
        return 0

    lax.fori_loop(0, (NCHUNK - 5) // NB, ring, 0)
    for i in range(NCHUNK - 5, NCHUNK):
        b = i % NB
        gwait(b)
        scat(b)
        if i + 2 < NCHUNK:
            b2 = (i + 2) % NB
            iwait(b2)
            gath(b2)
        swait(b)
        if i + NB < NCHUNK:
            idx_start(b, i + NB)
    plsc.subcore_barrier()

    pltpu.sync_copy(
        acc.at[pl.ds(s * RPT, RPT)],
        accp_hbm.at[pl.ds(c * NP + s * RPT, RPT)],
    )


def _comb_body(accp_ref, xs_ref, degp_ref, b_ref, o_ref):
    deg = degp_ref[0:N, 0:1] + degp_ref[NP : NP + N, 0:1] + 1.0
    dinv = lax.rsqrt(deg)
    o_ref[...] = (
        dinv * (accp_ref[0:N, :] + accp_ref[NP : NP + N, :] + xs_ref[...])
        + b_ref[...]
    )


_comb_kernel = pl.pallas_call(
    _comb_body,
    out_shape=jax.ShapeDtypeStruct((N, D), jnp.float32),
)


def kernel(x, edge_index, W, b):
    src = edge_index[0]
    dst = edge_index[1]
    dst3 = dst.reshape(NW, NCD, KD)
    degp = _deg_kernel(dst3)
    xs = _xw_kernel(x, W, degp)
    accp = _edge_kernel(xs, src, dst)
    return _comb_kernel(accp, xs, degp, b.reshape(1, D))


# speedup vs baseline: 47.9469x; 1.0160x over previous
import functools

import jax
import jax.numpy as jnp
from jax import lax
from jax.experimental import pallas as pl
from jax.experimental.pallas import tpu as pltpu
from jax.experimental.pallas import tpu_sc as plsc

N = 10000
E = 320000
D = 128

NC = 2
NS = 16
NW = NC * NS
EPW = E // NW
NP = 10240
RPT = NP // NS

KD = 125
NCD = EPW // KD
NBD = 4
NRD = NCD // NBD

K = 80
NCHUNK = EPW // K
NB = 4
NRINGS = (NCHUNK - 1) // NB

_mesh = plsc.VectorSubcoreMesh(core_axis_name="c", subcore_axis_name="s")


@functools.partial(
    pl.kernel,
    out_type=jax.ShapeDtypeStruct((NC * NP, 16), jnp.float32),
    mesh=_mesh,
    scratch_types=[
        pltpu.VMEM((NCD, KD), jnp.int32),
        pltpu.VMEM((KD, 16), jnp.float32),
        pltpu.VMEM((128, 16), jnp.float32),
        pltpu.VMEM_SHARED((NP, 16), jnp.float32),
    ]
    + [pltpu.SemaphoreType.DMA] * (NBD + 1),
)
def _deg_kernel(dst_hbm, degp_hbm, dstb, ones, zbuf, deg2, *sems):
    ssem = sems[:NBD]
    isem = sems[NBD]
    c = lax.axis_index("c")
    s = lax.axis_index("s")
    wid = c * NS + s

    pltpu.async_copy(dst_hbm.at[wid], dstb, isem)

    def fill(i, _):
        ones[i, :] = jnp.ones((16,), jnp.float32)
        return 0

    lax.fori_loop(0, KD, fill, 0)

    def fillz(i, _):
        zbuf[i, :] = jnp.zeros((16,), jnp.float32)
        return 0

    lax.fori_loop(0, 128, fillz, 0)

    for j in range(5):
        pltpu.sync_copy(zbuf, deg2.at[pl.ds(s * RPT + j * 128, 128)])
    pltpu.make_async_copy(dst_hbm.at[wid], dstb, isem).wait()
    plsc.subcore_barrier()

    def scat(b, i):
        pltpu.async_copy(ones, deg2.at[dstb.at[i]], ssem[b])

    def swait(b):
        pltpu.make_async_copy(ones, deg2.at[dstb.at[0]], ssem[b]).wait()

    for b in range(NBD):
        scat(b, b)

    def ring(g, _):
        for b in range(NBD):
            swait(b)
            scat(b, (g + 1) * NBD + b)
        return 0

    lax.fori_loop(0, NRD - 1, ring, 0)
    for b in range(NBD):
        swait(b)
    plsc.subcore_barrier()

    pltpu.sync_copy(
        deg2.at[pl.ds(s * RPT, RPT)],
        degp_hbm.at[pl.ds(c * NP + s * RPT, RPT)],
    )


def _xw_body(x_ref, w_ref, degp_ref, xs_ref):
    deg = degp_ref[0:N, 0:1] + degp_ref[NP : NP + N, 0:1] + 1.0
    dinv = lax.rsqrt(deg)
    xw = jnp.dot(x_ref[...], w_ref[...], preferred_element_type=jnp.float32)
    xs_ref[...] = dinv * xw


_xw_kernel = pl.pallas_call(
    _xw_body,
    out_shape=jax.ShapeDtypeStruct((N, D), jnp.float32),
)


@functools.partial(
    pl.kernel,
    out_type=jax.ShapeDtypeStruct((NC * NP, D), jnp.float32),
    mesh=_mesh,
    scratch_types=[pltpu.VMEM((K,), jnp.int32)] * NB
    + [pltpu.VMEM((K,), jnp.int32)] * NB
    + [pltpu.VMEM((K, D), jnp.float32)] * NB
    + [pltpu.VMEM_SHARED((NP, D), jnp.float32)]
    + [pltpu.SemaphoreType.DMA] * (3 * NB),
)
def _edge_kernel(xs_hbm, src_hbm, dst_hbm, accp_hbm, *rest):
    srcb = rest[:NB]
    dstb = rest[NB : 2 * NB]
    rows = rest[2 * NB : 3 * NB]
    acc = rest[3 * NB]
    isem = rest[3 * NB + 1 : 3 * NB + 1 + NB]
    gsem = rest[3 * NB + 1 + NB : 3 * NB + 1 + 2 * NB]
    ssem = rest[3 * NB + 1 + 2 * NB :]
    c = lax.axis_index("c")
    s = lax.axis_index("s")
    wid = c * NS + s
    base = wid * EPW

    def idx_start(b, i):
        off = base + i * K
        pltpu.async_copy(src_hbm.at[pl.ds(off, K)], srcb[b], isem[b])
        pltpu.async_copy(dst_hbm.at[pl.ds(off, K)], dstb[b], isem[b])

    def iwait(b):
        pltpu.make_async_copy(src_hbm.at[pl.ds(base, K)], srcb[b],
                              isem[b]).wait()
        pltpu.make_async_copy(dst_hbm.at[pl.ds(base, K)], dstb[b],
                              isem[b]).wait()

    def gath(b):
        pltpu.async_copy(xs_hbm.at[srcb[b]], rows[b], gsem[b])

    def gwait(b):
        pltpu.make_async_copy(xs_hbm.at[srcb[b]], rows[b], gsem[b]).wait()

    def scat(b):
        pltpu.async_copy(rows[b], acc.at[dstb[b]], ssem[b])

    def swait(b):
        pltpu.make_async_copy(rows[b], acc.at[dstb[b]], ssem[b]).wait()

    for b in range(NB):
        idx_start(b, b)
    iwait(0)
    gath(0)
    iwait(1)
    gath(1)

    def zrows(i, _):
        rows[NB - 1][i // 8, pl.ds((i % 8) * 16, 16)] = jnp.zeros(
            (16,), jnp.float32
        )
        return 0

    lax.fori_loop(0, K * 8, zrows, 0)
    for j in range(8):
        pltpu.sync_copy(rows[NB - 1], acc.at[pl.ds(s * RPT + j * K, K)])
    plsc.subcore_barrier()

    def ring(g, _):
        for b in range(NB):
            i = g * NB + b
            gwait(b)
            scat(b)
            b2 = (b + 2) % NB
            iwait(b2)
            gath(b2)
            swait(b)
            idx_start(b, i + NB)
        return 0

    lax.fori_loop(0, (NCHUNK - 5) // NB, ring, 0)
    for i in range(NCHUNK - 5, NCHUNK):
        b = i % NB
        gwait(b)
        scat(b)
        if i + 2 < NCHUNK:
            b2 = (i + 2) % NB
            iwait(b2)
            gath(b2)
        swait(b)
        if i + NB < NCHUNK:
            idx_start(b, i + NB)
    plsc.subcore_barrier()

    pltpu.sync_copy(
        acc.at[pl.ds(s * RPT, RPT)],
        accp_hbm.at[pl.ds(c * NP + s * RPT, RPT)],
    )


def _comb_body(accp_ref, xs_ref, degp_ref, b_ref, o_ref):
    deg = degp_ref[0:N, 0:1] + degp_ref[NP : NP + N, 0:1] + 1.0
    dinv = lax.rsqrt(deg)
    o_ref[...] = (
        dinv * (accp_ref[0:N, :] + accp_ref[NP : NP + N, :] + xs_ref[...])
        + b_ref[...]
    )


_comb_kernel = pl.pallas_call(
    _comb_body,
    out_shape=jax.ShapeDtypeStruct((N, D), jnp.float32),
)


def kernel(x, edge_index, W, b):
    src = edge_index[0]
    dst = edge_index[1]
    dst3 = dst.reshape(NW, NCD, KD)
    degp = _deg_kernel(dst3)
    xs = _xw_kernel(x, W, degp)
    accp = _edge_kernel(xs, src, dst)
    return _comb_kernel(accp, xs, degp, b.reshape(1, D))
